# Initial kernel scaffold; baseline (speedup 1.0000x reference)
import jax
import jax.numpy as jnp
from jax.experimental import pallas as pl


def kernel(pos, batch, params):
    raise NotImplementedError("write your pallas kernel here")


# speedup vs baseline: 4.5233x; 4.5233x over previous
import functools

import jax
import jax.numpy as jnp
from jax import lax
from jax.experimental import pallas as pl
from jax.experimental.pallas import tpu as pltpu
from jax.experimental.pallas import tpu_sc as plsc

_DEC = 4
_K = 16
_BLOCKS = [(3, 32), (32, 128), (128, 256), (256, 512)]
_BIG = 1e30


def _leaky(x, s):
    return jnp.where(x >= 0, x, s * x)


def _pad_lanes(x, w):
    if x.shape[1] == w:
        return x
    return jnp.pad(x, ((0, 0), (0, w - x.shape[1])))


def _pad_rows(x, n):
    if x.shape[0] == n:
        return x
    return jnp.pad(x, ((0, n - x.shape[0]), (0, 0)))


def _knn_body(pq_ref, pt_ref, col_ref, *, n):
    pq = pq_ref[...]
    pt = pt_ref[...]
    d = ((pq[:, 0:1] - pt[0:1, :]) ** 2
         + (pq[:, 1:2] - pt[1:2, :]) ** 2
         + (pq[:, 2:3] - pt[2:3, :]) ** 2)
    d = jnp.where(pq[:, 3:4] != pt[3:4, :], _BIG, d)
    iota = lax.broadcasted_iota(jnp.int32, d.shape, 1)
    liota = lax.broadcasted_iota(jnp.int32, (d.shape[0], _K), 1)
    acc = jnp.zeros((d.shape[0], _K), jnp.int32)
    for t in range(_K):
        m = jnp.min(d, axis=1, keepdims=True)
        am = jnp.min(jnp.where(d == m, iota, n), axis=1, keepdims=True)
        acc = jnp.where(liota == t, am, acc)
        d = jnp.where(iota == am, jnp.inf, d)
    col_ref[...] = acc


def _knn(pq, pt):
    nq = pq.shape[0]
    n = pt.shape[1]
    tq = min(nq, 128)
    grid = nq // tq
    return pl.pallas_call(
        functools.partial(_knn_body, n=n),
        grid=(grid,),
        in_specs=[
            pl.BlockSpec((tq, 8), lambda i: (i, 0)),
            pl.BlockSpec((8, n), lambda i: (0, 0)),
        ],
        out_specs=pl.BlockSpec((tq, _K), lambda i: (i, 0)),
        out_shape=jax.ShapeDtypeStruct((nq, _K), jnp.int32),
    )(pq, pt)


def _h1_body(x_ref, w_ref, b_ref, o_ref):
    o_ref[...] = _leaky(
        jnp.dot(x_ref[...], w_ref[...], preferred_element_type=jnp.float32)
        + b_ref[...], 0.2)


def _h1(x, w, b):
    n = x.shape[0]
    return pl.pallas_call(
        _h1_body,
        out_shape=jax.ShapeDtypeStruct((n, w.shape[1]), jnp.float32),
    )(x, w, b)


def _sc_gather(table, idx):
    b = idx.shape[0]
    d = table.shape[1]
    info = plsc.get_sparse_core_info()
    ncores = info.num_cores
    nw_max = info.num_cores * info.num_subcores
    nw = min(nw_max, b // 8)
    c = b // nw
    s = min(c, 128)
    nloop = c // s
    mesh = plsc.VectorSubcoreMesh(core_axis_name="c", subcore_axis_name="s")

    @functools.partial(
        pl.kernel, mesh=mesh,
        out_type=jax.ShapeDtypeStruct((b, d), jnp.float32),
        scratch_types=[
            pltpu.VMEM((c,), jnp.int32),
            pltpu.VMEM((s, d), jnp.float32),
            pltpu.VMEM((s, d), jnp.float32),
            pltpu.SemaphoreType.DMA,
            pltpu.SemaphoreType.DMA,
        ],
    )
    def k(table_hbm, idx_hbm, out_hbm, idx_v, row0, row1, sem0, sem1):
        wid = lax.axis_index("s") * ncores + lax.axis_index("c")
        bufs = (row0, row1)
        sems = (sem0, sem1)

        @pl.when(wid < nw)
        def _():
            base = wid * c
            pltpu.sync_copy(idx_hbm.at[pl.ds(base, c)], idx_v)
            pend = [None, None]
            pend[0] = pltpu.async_copy(
                table_hbm.at[idx_v.at[pl.ds(0, s)]], bufs[0], sems[0])
            for j in range(nloop):
                cur = j % 2
                nxt = (j + 1) % 2
                if j + 1 < nloop:
                    pend[nxt] = pltpu.async_copy(
                        table_hbm.at[idx_v.at[pl.ds((j + 1) * s, s)]],
                        bufs[nxt], sems[nxt])
                pend[cur].wait()
                pltpu.sync_copy(bufs[cur], out_hbm.at[pl.ds(base + j * s, s)])

    return k(table, idx)


def _lfa_body(e_ref, pq_ref, wenc_ref, benc_ref,
              watt_ref, batt_ref, o_ref, *, dj):
    e = e_ref[...]
    te = e.shape[0]
    tq = pq_ref.shape[0]
    f32 = jnp.float32
    hi = lax.Precision.HIGHEST
    r_eq = lax.broadcasted_iota(jnp.int32, (te, tq), 0) // _K
    r_q = lax.broadcasted_iota(jnp.int32, (te, tq), 1)
    rmat = (r_eq == r_q).astype(f32)
    pos_i = jnp.dot(rmat, pq_ref[...], preferred_element_type=f32, precision=hi)
    pos_j = e[:, 0:8]
    x_j = e[:, 8:8 + dj]
    dist = pos_j - pos_i
    euclid = jnp.sum(jnp.sqrt(dist * dist), axis=1, keepdims=True)
    rel = jnp.concatenate(
        [pos_i[:, 0:3], pos_j[:, 0:3], dist[:, 0:3], euclid,
         jnp.zeros((te, 6), f32)], axis=1)
    enc = jnp.dot(rel, wenc_ref[...], preferred_element_type=f32) + benc_ref[...]
    out1 = jnp.concatenate([x_j, enc], axis=1)
    lo = jnp.dot(out1, watt_ref[...], preferred_element_type=f32) + batt_ref[...]
    lo = lo - jnp.max(lo, axis=1, keepdims=True)
    p = jnp.exp(lo)
    att = p / jnp.sum(p, axis=1, keepdims=True)
    msg = att * out1
    r2_q = lax.broadcasted_iota(jnp.int32, (tq, te), 0)
    r2_e = lax.broadcasted_iota(jnp.int32, (tq, te), 1) // _K
    r2 = (r2_q == r2_e).astype(f32)
    o_ref[...] = jnp.dot(r2, msg, preferred_element_type=f32, precision=hi)


def _lfa(e, pq, wenc, benc, watt, batt):
    nq = pq.shape[0]
    w = e.shape[1]
    dj = watt.shape[0] - wenc.shape[1]
    d1 = watt.shape[0]
    tq = min(nq, 128)
    te = tq * _K
    grid = nq // tq
    full = lambda a: pl.BlockSpec(a.shape, lambda i: tuple(0 for _ in a.shape))
    return pl.pallas_call(
        functools.partial(_lfa_body, dj=dj),
        grid=(grid,),
        in_specs=[
            pl.BlockSpec((te, w), lambda i: (i, 0)),
            pl.BlockSpec((tq, 8), lambda i: (i, 0)),
            full(wenc), full(benc), full(watt), full(batt),
        ],
        out_specs=pl.BlockSpec((tq, d1), lambda i: (i, 0)),
        out_shape=jax.ShapeDtypeStruct((nq, d1), jnp.float32),
    )(e, pq, wenc, benc, watt, batt)


def _out_body(a2_ref, x4_ref, w2_ref, b2_ref, wsc_ref, bsc_ref, o_ref):
    f32 = jnp.float32
    h2 = _leaky(jnp.dot(a2_ref[...], w2_ref[...], preferred_element_type=f32)
                + b2_ref[...], 0.2)
    sc = _leaky(jnp.dot(x4_ref[...], wsc_ref[...], preferred_element_type=f32)
                + bsc_ref[...], 0.2)
    o_ref[...] = _leaky(h2 + sc, 0.01)


def _block_out(a2, x4, w2, b2, wsc, bsc):
    m = a2.shape[0]
    return pl.pallas_call(
        _out_body,
        out_shape=jax.ShapeDtypeStruct((m, w2.shape[1]), jnp.float32),
    )(a2, x4, w2, b2, wsc, bsc)


def _head_body(x_ref, bcol_ref, wp_ref, bp_ref, w1_ref, b1_ref, g1_ref,
               e1_ref, w2_ref, b2_ref, g2_ref, e2_ref, w3_ref, b3_ref, o_ref):
    f32 = jnp.float32
    xp = jnp.dot(x_ref[...], wp_ref[...], preferred_element_type=f32) + bp_ref[...]
    bcol = bcol_ref[...]
    neg = jnp.float32(-jnp.inf)
    m0 = jnp.max(jnp.where(bcol == 0.0, xp, neg), axis=0, keepdims=True)
    m1 = jnp.max(jnp.where(bcol == 1.0, xp, neg), axis=0, keepdims=True)
    h = jnp.concatenate([m0, m1], axis=0)

    def bn_relu(z, g, be):
        mu = jnp.mean(z, axis=0, keepdims=True)
        va = jnp.mean((z - mu) ** 2, axis=0, keepdims=True)
        return jnp.maximum((z - mu) / jnp.sqrt(va + 1e-5) * g + be, 0.0)

    h = bn_relu(jnp.dot(h, w1_ref[...], preferred_element_type=f32) + b1_ref[...],
                g1_ref[...], e1_ref[...])
    h = bn_relu(jnp.dot(h, w2_ref[...], preferred_element_type=f32) + b2_ref[...],
                g2_ref[...], e2_ref[...])
    o = jnp.dot(h, w3_ref[...], preferred_element_type=f32) + b3_ref[...]
    o = o - jnp.max(o, axis=1, keepdims=True)
    o_ref[...] = o - jnp.log(jnp.sum(jnp.exp(o), axis=1, keepdims=True))


def _head(x, bcol, p):
    args = (x, bcol, p['w_pool'], p['b_pool'].reshape(1, -1),
            p['w_f1'], p['b_f1'].reshape(1, -1),
            p['g_f1'].reshape(1, -1), p['be_f1'].reshape(1, -1),
            p['w_f2'], p['b_f2'].reshape(1, -1),
            p['g_f2'].reshape(1, -1), p['be_f2'].reshape(1, -1),
            p['w_f3'], p['b_f3'].reshape(1, -1))
    return pl.pallas_call(
        _head_body,
        out_shape=jax.ShapeDtypeStruct((2, 10), jnp.float32),
    )(*args)


def _round128(v):
    return (v + 127) // 128 * 128


def kernel(pos, batch, params):
    p = params
    x = pos
    bf = batch.astype(jnp.float32)
    for i, (din, dout) in enumerate(_BLOCKS):
        n = x.shape[0]
        nq = n // _DEC
        d4 = dout // 4
        d2 = dout // 2
        posp = _pad_lanes(pos, 8)
        pqk = posp.at[:, 3].set(bf)[::_DEC]
        ptk = posp.at[:, 3].set(bf).T
        col = _knn(pqk, ptk)

        dinp = max(8, din)
        xw = _pad_lanes(x, dinp)
        w1 = _pad_rows(p['w1_%d' % i], dinp)
        h1 = _h1(xw, w1, p['b1_%d' % i].reshape(1, -1))

        w1t = _round128(8 + d4)
        table1 = _pad_lanes(jnp.concatenate([posp, h1], axis=1), w1t)
        e1 = _sc_gather(table1, col.reshape(-1))
        wenc = _pad_rows(p['wenc1_%d' % i], 16)
        a1 = _lfa(e1, posp[:nq],
                  wenc, p['benc1_%d' % i].reshape(1, -1),
                  p['watt1_%d' % i], p['batt1_%d' % i].reshape(1, -1))

        w2t = _round128(8 + d2)
        table2 = _pad_lanes(
            jnp.concatenate([posp, _pad_rows(a1, n)], axis=1), w2t)
        col2 = col[::_DEC].reshape(-1)
        e2 = _sc_gather(table2, col2)
        wenc = _pad_rows(p['wenc2_%d' % i], 16)
        pq2 = posp[::_DEC][:nq // _DEC]
        a2q = _lfa(e2, pq2,
                   wenc, p['benc2_%d' % i].reshape(1, -1),
                   p['watt2_%d' % i], p['batt2_%d' % i].reshape(1, -1))

        a2 = _pad_rows(a2q, nq)
        x4 = xw[::_DEC]
        wsc = _pad_rows(p['wsc_%d' % i], dinp)
        x = _block_out(a2, x4, p['w2_%d' % i], p['b2_%d' % i].reshape(1, -1),
                       wsc, p['bsc_%d' % i].reshape(1, -1))
        pos = pos[::_DEC]
        batch = batch[::_DEC]
        bf = bf[::_DEC]

    bcol = bf.reshape(-1, 1)
    return _head(x, bcol, p)


# speedup vs baseline: 4.7669x; 1.0538x over previous
import functools

import jax
import jax.numpy as jnp
from jax import lax
from jax.experimental import pallas as pl
from jax.experimental.pallas import tpu as pltpu
from jax.experimental.pallas import tpu_sc as plsc

_DEC = 4
_K = 16
_BLOCKS = [(3, 32), (32, 128), (128, 256), (256, 512)]
_BIG = 1e30


def _leaky(x, s):
    return jnp.where(x >= 0, x, s * x)


def _pad_lanes(x, w):
    if x.shape[1] == w:
        return x
    return jnp.pad(x, ((0, 0), (0, w - x.shape[1])))


def _pad_rows(x, n):
    if x.shape[0] == n:
        return x
    return jnp.pad(x, ((0, n - x.shape[0]), (0, 0)))


def _knn_body(pq_ref, pt_ref, col_ref, *, n):
    pq = pq_ref[...]
    pt = pt_ref[...]
    d = ((pq[:, 0:1] - pt[0:1, :]) ** 2
         + (pq[:, 1:2] - pt[1:2, :]) ** 2
         + (pq[:, 2:3] - pt[2:3, :]) ** 2)
    d = jnp.where(pq[:, 3:4] != pt[3:4, :], _BIG, d)
    iota = lax.broadcasted_iota(jnp.int32, d.shape, 1)
    liota = lax.broadcasted_iota(jnp.int32, (d.shape[0], _K), 1)
    acc = jnp.zeros((d.shape[0], _K), jnp.int32)
    for t in range(_K):
        am = jnp.argmin(d, axis=1).astype(jnp.int32)[:, None]
        acc = jnp.where(liota == t, am, acc)
        d = jnp.where(iota == am, jnp.inf, d)
    col_ref[...] = acc


def _knn(pq, pt):
    nq = pq.shape[0]
    n = pt.shape[1]
    tq = min(nq, 128)
    grid = nq // tq
    return pl.pallas_call(
        functools.partial(_knn_body, n=n),
        grid=(grid,),
        in_specs=[
            pl.BlockSpec((tq, 8), lambda i: (i, 0)),
            pl.BlockSpec((8, n), lambda i: (0, 0)),
        ],
        out_specs=pl.BlockSpec((tq, _K), lambda i: (i, 0)),
        out_shape=jax.ShapeDtypeStruct((nq, _K), jnp.int32),
    )(pq, pt)


def _h1_body(x_ref, w_ref, b_ref, o_ref):
    o_ref[...] = _leaky(
        jnp.dot(x_ref[...], w_ref[...], preferred_element_type=jnp.float32)
        + b_ref[...], 0.2)


def _h1(x, w, b):
    n = x.shape[0]
    return pl.pallas_call(
        _h1_body,
        out_shape=jax.ShapeDtypeStruct((n, w.shape[1]), jnp.float32),
    )(x, w, b)


def _sc_gather(table, idx):
    b = idx.shape[0]
    d = table.shape[1]
    info = plsc.get_sparse_core_info()
    ncores = info.num_cores
    nw_max = info.num_cores * info.num_subcores
    nw = min(nw_max, b // 8)
    c = b // nw
    s = min(c, 128)
    nloop = c // s
    mesh = plsc.VectorSubcoreMesh(core_axis_name="c", subcore_axis_name="s")

    @functools.partial(
        pl.kernel, mesh=mesh,
        out_type=jax.ShapeDtypeStruct((b, d), jnp.float32),
        scratch_types=[
            pltpu.VMEM((c,), jnp.int32),
            pltpu.VMEM((s, d), jnp.float32),
            pltpu.VMEM((s, d), jnp.float32),
            pltpu.SemaphoreType.DMA,
            pltpu.SemaphoreType.DMA,
        ],
    )
    def k(table_hbm, idx_hbm, out_hbm, idx_v, row0, row1, sem0, sem1):
        wid = lax.axis_index("s") * ncores + lax.axis_index("c")
        bufs = (row0, row1)
        sems = (sem0, sem1)

        @pl.when(wid < nw)
        def _():
            base = wid * c
            pltpu.sync_copy(idx_hbm.at[pl.ds(base, c)], idx_v)
            pend = [None, None]
            pend[0] = pltpu.async_copy(
                table_hbm.at[idx_v.at[pl.ds(0, s)]], bufs[0], sems[0])
            for j in range(nloop):
                cur = j % 2
                nxt = (j + 1) % 2
                if j + 1 < nloop:
                    pend[nxt] = pltpu.async_copy(
                        table_hbm.at[idx_v.at[pl.ds((j + 1) * s, s)]],
                        bufs[nxt], sems[nxt])
                pend[cur].wait()
                pltpu.sync_copy(bufs[cur], out_hbm.at[pl.ds(base + j * s, s)])

    return k(table, idx)


def _lfa_body(e_ref, pq_ref, wenc_ref, benc_ref,
              watt_ref, batt_ref, o_ref, *, dj):
    e = e_ref[...]
    te = e.shape[0]
    tq = pq_ref.shape[0]
    f32 = jnp.float32
    hi = lax.Precision.HIGHEST
    r_eq = lax.broadcasted_iota(jnp.int32, (te, tq), 0) // _K
    r_q = lax.broadcasted_iota(jnp.int32, (te, tq), 1)
    rmat = (r_eq == r_q).astype(f32)
    pos_i = jnp.dot(rmat, pq_ref[...], preferred_element_type=f32, precision=hi)
    pos_j = e[:, 0:8]
    x_j = e[:, 8:8 + dj]
    dist = pos_j - pos_i
    euclid = jnp.sum(jnp.sqrt(dist * dist), axis=1, keepdims=True)
    rel = jnp.concatenate(
        [pos_i[:, 0:3], pos_j[:, 0:3], dist[:, 0:3], euclid,
         jnp.zeros((te, 6), f32)], axis=1)
    enc = jnp.dot(rel, wenc_ref[...], preferred_element_type=f32) + benc_ref[...]
    out1 = jnp.concatenate([x_j, enc], axis=1)
    lo = jnp.dot(out1, watt_ref[...], preferred_element_type=f32) + batt_ref[...]
    lo = lo - jnp.max(lo, axis=1, keepdims=True)
    p = jnp.exp(lo)
    att = p / jnp.sum(p, axis=1, keepdims=True)
    msg = att * out1
    r2_q = lax.broadcasted_iota(jnp.int32, (tq, te), 0)
    r2_e = lax.broadcasted_iota(jnp.int32, (tq, te), 1) // _K
    r2 = (r2_q == r2_e).astype(f32)
    o_ref[...] = jnp.dot(r2, msg, preferred_element_type=f32, precision=hi)


def _lfa(e, pq, wenc, benc, watt, batt):
    nq = pq.shape[0]
    w = e.shape[1]
    dj = watt.shape[0] - wenc.shape[1]
    d1 = watt.shape[0]
    tq = min(nq, 128)
    te = tq * _K
    grid = nq // tq
    full = lambda a: pl.BlockSpec(a.shape, lambda i: tuple(0 for _ in a.shape))
    return pl.pallas_call(
        functools.partial(_lfa_body, dj=dj),
        grid=(grid,),
        in_specs=[
            pl.BlockSpec((te, w), lambda i: (i, 0)),
            pl.BlockSpec((tq, 8), lambda i: (i, 0)),
            full(wenc), full(benc), full(watt), full(batt),
        ],
        out_specs=pl.BlockSpec((tq, d1), lambda i: (i, 0)),
        out_shape=jax.ShapeDtypeStruct((nq, d1), jnp.float32),
    )(e, pq, wenc, benc, watt, batt)


def _out_body(a2_ref, x4_ref, w2_ref, b2_ref, wsc_ref, bsc_ref, o_ref):
    f32 = jnp.float32
    h2 = _leaky(jnp.dot(a2_ref[...], w2_ref[...], preferred_element_type=f32)
                + b2_ref[...], 0.2)
    sc = _leaky(jnp.dot(x4_ref[...], wsc_ref[...], preferred_element_type=f32)
                + bsc_ref[...], 0.2)
    o_ref[...] = _leaky(h2 + sc, 0.01)


def _block_out(a2, x4, w2, b2, wsc, bsc):
    m = a2.shape[0]
    return pl.pallas_call(
        _out_body,
        out_shape=jax.ShapeDtypeStruct((m, w2.shape[1]), jnp.float32),
    )(a2, x4, w2, b2, wsc, bsc)


def _head_body(x_ref, bcol_ref, wp_ref, bp_ref, w1_ref, b1_ref, g1_ref,
               e1_ref, w2_ref, b2_ref, g2_ref, e2_ref, w3_ref, b3_ref, o_ref):
    f32 = jnp.float32
    xp = jnp.dot(x_ref[...], wp_ref[...], preferred_element_type=f32) + bp_ref[...]
    bcol = bcol_ref[...]
    neg = jnp.float32(-jnp.inf)
    m0 = jnp.max(jnp.where(bcol == 0.0, xp, neg), axis=0, keepdims=True)
    m1 = jnp.max(jnp.where(bcol == 1.0, xp, neg), axis=0, keepdims=True)
    h = jnp.concatenate([m0, m1], axis=0)

    def bn_relu(z, g, be):
        mu = jnp.mean(z, axis=0, keepdims=True)
        va = jnp.mean((z - mu) ** 2, axis=0, keepdims=True)
        return jnp.maximum((z - mu) / jnp.sqrt(va + 1e-5) * g + be, 0.0)

    h = bn_relu(jnp.dot(h, w1_ref[...], preferred_element_type=f32) + b1_ref[...],
                g1_ref[...], e1_ref[...])
    h = bn_relu(jnp.dot(h, w2_ref[...], preferred_element_type=f32) + b2_ref[...],
                g2_ref[...], e2_ref[...])
    o = jnp.dot(h, w3_ref[...], preferred_element_type=f32) + b3_ref[...]
    o = o - jnp.max(o, axis=1, keepdims=True)
    o_ref[...] = o - jnp.log(jnp.sum(jnp.exp(o), axis=1, keepdims=True))


def _head(x, bcol, p):
    args = (x, bcol, p['w_pool'], p['b_pool'].reshape(1, -1),
            p['w_f1'], p['b_f1'].reshape(1, -1),
            p['g_f1'].reshape(1, -1), p['be_f1'].reshape(1, -1),
            p['w_f2'], p['b_f2'].reshape(1, -1),
            p['g_f2'].reshape(1, -1), p['be_f2'].reshape(1, -1),
            p['w_f3'], p['b_f3'].reshape(1, -1))
    return pl.pallas_call(
        _head_body,
        out_shape=jax.ShapeDtypeStruct((2, 10), jnp.float32),
    )(*args)


def _round128(v):
    return (v + 127) // 128 * 128


def kernel(pos, batch, params):
    p = params
    x = pos
    bf = batch.astype(jnp.float32)
    for i, (din, dout) in enumerate(_BLOCKS):
        n = x.shape[0]
        nq = n // _DEC
        d4 = dout // 4
        d2 = dout // 2
        posp = _pad_lanes(pos, 8)
        pqk = posp.at[:, 3].set(bf)[::_DEC]
        ptk = posp.at[:, 3].set(bf).T
        col = _knn(pqk, ptk)

        dinp = max(8, din)
        xw = _pad_lanes(x, dinp)
        w1 = _pad_rows(p['w1_%d' % i], dinp)
        h1 = _h1(xw, w1, p['b1_%d' % i].reshape(1, -1))

        w1t = _round128(8 + d4)
        table1 = _pad_lanes(jnp.concatenate([posp, h1], axis=1), w1t)
        e1 = _sc_gather(table1, col.reshape(-1))
        wenc = _pad_rows(p['wenc1_%d' % i], 16)
        a1 = _lfa(e1, posp[:nq],
                  wenc, p['benc1_%d' % i].reshape(1, -1),
                  p['watt1_%d' % i], p['batt1_%d' % i].reshape(1, -1))

        w2t = _round128(8 + d2)
        table2 = _pad_lanes(
            jnp.concatenate([posp, _pad_rows(a1, n)], axis=1), w2t)
        col2 = col[::_DEC].reshape(-1)
        e2 = _sc_gather(table2, col2)
        wenc = _pad_rows(p['wenc2_%d' % i], 16)
        pq2 = posp[::_DEC][:nq // _DEC]
        a2q = _lfa(e2, pq2,
                   wenc, p['benc2_%d' % i].reshape(1, -1),
                   p['watt2_%d' % i], p['batt2_%d' % i].reshape(1, -1))

        a2 = _pad_rows(a2q, nq)
        x4 = xw[::_DEC]
        wsc = _pad_rows(p['wsc_%d' % i], dinp)
        x = _block_out(a2, x4, p['w2_%d' % i], p['b2_%d' % i].reshape(1, -1),
                       wsc, p['bsc_%d' % i].reshape(1, -1))
        pos = pos[::_DEC]
        batch = batch[::_DEC]
        bf = bf[::_DEC]

    bcol = bf.reshape(-1, 1)
    return _head(x, bcol, p)


# speedup vs baseline: 4.9459x; 1.0376x over previous
import functools

import jax
import jax.numpy as jnp
from jax import lax
from jax.experimental import pallas as pl
from jax.experimental.pallas import tpu as pltpu
from jax.experimental.pallas import tpu_sc as plsc

_DEC = 4
_K = 16
_BLOCKS = [(3, 32), (32, 128), (128, 256), (256, 512)]
_BIG = 1e30


def _leaky(x, s):
    return jnp.where(x >= 0, x, s * x)


def _pad_lanes(x, w):
    if x.shape[1] == w:
        return x
    return jnp.pad(x, ((0, 0), (0, w - x.shape[1])))


def _pad_rows(x, n):
    if x.shape[0] == n:
        return x
    return jnp.pad(x, ((0, n - x.shape[0]), (0, 0)))


def _knn_h1_body(pq_ref, pt_ref, x_ref, posp_ref, w1_ref, b1_ref,
                 col_ref, tab_ref, *, n, w1t):
    pq = pq_ref[...]
    pt = pt_ref[...]
    d = ((pq[:, 0:1] - pt[0:1, :]) ** 2
         + (pq[:, 1:2] - pt[1:2, :]) ** 2
         + (pq[:, 2:3] - pt[2:3, :]) ** 2)
    d = jnp.where(pq[:, 3:4] != pt[3:4, :], _BIG, d)
    iota = lax.broadcasted_iota(jnp.int32, d.shape, 1)
    liota = lax.broadcasted_iota(jnp.int32, (d.shape[0], _K), 1)
    acc = jnp.zeros((d.shape[0], _K), jnp.int32)
    for t in range(_K):
        am = jnp.argmin(d, axis=1).astype(jnp.int32)[:, None]
        acc = jnp.where(liota == t, am, acc)
        d = jnp.where(iota == am, jnp.inf, d)
    col_ref[...] = acc
    h1 = _leaky(
        jnp.dot(x_ref[...], w1_ref[...], preferred_element_type=jnp.float32)
        + b1_ref[...], 0.2)
    rows = h1.shape[0]
    pad = w1t - 8 - h1.shape[1]
    tab_ref[...] = jnp.concatenate(
        [posp_ref[...], h1, jnp.zeros((rows, pad), jnp.float32)], axis=1)


def _knn_h1(pq, pt, x, posp, w1, b1, w1t):
    nq = pq.shape[0]
    n = pt.shape[1]
    tq = min(nq, 128)
    grid = nq // tq
    tr = n // grid
    d4 = w1.shape[1]
    full = lambda a: pl.BlockSpec(a.shape, lambda i: tuple(0 for _ in a.shape))
    return pl.pallas_call(
        functools.partial(_knn_h1_body, n=n, w1t=w1t),
        grid=(grid,),
        in_specs=[
            pl.BlockSpec((tq, 8), lambda i: (i, 0)),
            pl.BlockSpec((8, n), lambda i: (0, 0)),
            pl.BlockSpec((tr, x.shape[1]), lambda i: (i, 0)),
            pl.BlockSpec((tr, 8), lambda i: (i, 0)),
            full(w1), full(b1),
        ],
        out_specs=[
            pl.BlockSpec((tq, _K), lambda i: (i, 0)),
            pl.BlockSpec((tr, w1t), lambda i: (i, 0)),
        ],
        out_shape=[
            jax.ShapeDtypeStruct((nq, _K), jnp.int32),
            jax.ShapeDtypeStruct((n, w1t), jnp.float32),
        ],
    )(pq, pt, x, posp, w1, b1)


def _sc_gather(table, idx):
    b = idx.shape[0]
    d = table.shape[1]
    info = plsc.get_sparse_core_info()
    ncores = info.num_cores
    nw_max = info.num_cores * info.num_subcores
    nw = min(nw_max, b // 8)
    c = b // nw
    s = min(c, 128)
    nloop = c // s
    mesh = plsc.VectorSubcoreMesh(core_axis_name="c", subcore_axis_name="s")

    @functools.partial(
        pl.kernel, mesh=mesh,
        out_type=jax.ShapeDtypeStruct((b, d), jnp.float32),
        scratch_types=[
            pltpu.VMEM((c,), jnp.int32),
            pltpu.VMEM((s, d), jnp.float32),
            pltpu.VMEM((s, d), jnp.float32),
            pltpu.SemaphoreType.DMA,
            pltpu.SemaphoreType.DMA,
        ],
    )
    def k(table_hbm, idx_hbm, out_hbm, idx_v, row0, row1, sem0, sem1):
        wid = lax.axis_index("s") * ncores + lax.axis_index("c")
        bufs = (row0, row1)
        sems = (sem0, sem1)

        @pl.when(wid < nw)
        def _():
            base = wid * c
            pltpu.sync_copy(idx_hbm.at[pl.ds(base, c)], idx_v)
            pend = [None, None]
            pend[0] = pltpu.async_copy(
                table_hbm.at[idx_v.at[pl.ds(0, s)]], bufs[0], sems[0])
            for j in range(nloop):
                cur = j % 2
                nxt = (j + 1) % 2
                if j + 1 < nloop:
                    pend[nxt] = pltpu.async_copy(
                        table_hbm.at[idx_v.at[pl.ds((j + 1) * s, s)]],
                        bufs[nxt], sems[nxt])
                pend[cur].wait()
                pltpu.sync_copy(bufs[cur], out_hbm.at[pl.ds(base + j * s, s)])

    return k(table, idx)


def _lfa_body(e_ref, pq_ref, wenc_ref, benc_ref,
              watt_ref, batt_ref, o_ref, *, dj):
    e = e_ref[...]
    te = e.shape[0]
    tq = pq_ref.shape[0]
    f32 = jnp.float32
    hi = lax.Precision.HIGHEST
    r_eq = lax.broadcasted_iota(jnp.int32, (te, tq), 0) // _K
    r_q = lax.broadcasted_iota(jnp.int32, (te, tq), 1)
    rmat = (r_eq == r_q).astype(f32)
    pos_i = jnp.dot(rmat, pq_ref[...], preferred_element_type=f32, precision=hi)
    pos_j = e[:, 0:8]
    x_j = e[:, 8:8 + dj]
    dist = pos_j - pos_i
    euclid = jnp.sum(jnp.sqrt(dist * dist), axis=1, keepdims=True)
    rel = jnp.concatenate(
        [pos_i[:, 0:3], pos_j[:, 0:3], dist[:, 0:3], euclid,
         jnp.zeros((te, 6), f32)], axis=1)
    enc = jnp.dot(rel, wenc_ref[...], preferred_element_type=f32) + benc_ref[...]
    out1 = jnp.concatenate([x_j, enc], axis=1)
    lo = jnp.dot(out1, watt_ref[...], preferred_element_type=f32) + batt_ref[...]
    lo = lo - jnp.max(lo, axis=1, keepdims=True)
    p = jnp.exp(lo)
    att = p / jnp.sum(p, axis=1, keepdims=True)
    msg = att * out1
    r2_q = lax.broadcasted_iota(jnp.int32, (tq, te), 0)
    r2_e = lax.broadcasted_iota(jnp.int32, (tq, te), 1) // _K
    r2 = (r2_q == r2_e).astype(f32)
    o_ref[...] = jnp.dot(r2, msg, preferred_element_type=f32, precision=hi)


def _lfa(e, pq, wenc, benc, watt, batt):
    nq = pq.shape[0]
    w = e.shape[1]
    dj = watt.shape[0] - wenc.shape[1]
    d1 = watt.shape[0]
    tq = min(nq, 128)
    te = tq * _K
    grid = nq // tq
    full = lambda a: pl.BlockSpec(a.shape, lambda i: tuple(0 for _ in a.shape))
    return pl.pallas_call(
        functools.partial(_lfa_body, dj=dj),
        grid=(grid,),
        in_specs=[
            pl.BlockSpec((te, w), lambda i: (i, 0)),
            pl.BlockSpec((tq, 8), lambda i: (i, 0)),
            full(wenc), full(benc), full(watt), full(batt),
        ],
        out_specs=pl.BlockSpec((tq, d1), lambda i: (i, 0)),
        out_shape=jax.ShapeDtypeStruct((nq, d1), jnp.float32),
    )(e, pq, wenc, benc, watt, batt)


def _out_body(a2_ref, x4_ref, w2_ref, b2_ref, wsc_ref, bsc_ref, o_ref):
    f32 = jnp.float32
    h2 = _leaky(jnp.dot(a2_ref[...], w2_ref[...], preferred_element_type=f32)
                + b2_ref[...], 0.2)
    sc = _leaky(jnp.dot(x4_ref[...], wsc_ref[...], preferred_element_type=f32)
                + bsc_ref[...], 0.2)
    o_ref[...] = _leaky(h2 + sc, 0.01)


def _block_out(a2, x4, w2, b2, wsc, bsc):
    m = a2.shape[0]
    return pl.pallas_call(
        _out_body,
        out_shape=jax.ShapeDtypeStruct((m, w2.shape[1]), jnp.float32),
    )(a2, x4, w2, b2, wsc, bsc)


def _head_body(x_ref, bcol_ref, wp_ref, bp_ref, w1_ref, b1_ref, g1_ref,
               e1_ref, w2_ref, b2_ref, g2_ref, e2_ref, w3_ref, b3_ref, o_ref):
    f32 = jnp.float32
    xp = jnp.dot(x_ref[...], wp_ref[...], preferred_element_type=f32) + bp_ref[...]
    bcol = bcol_ref[...]
    neg = jnp.float32(-jnp.inf)
    m0 = jnp.max(jnp.where(bcol == 0.0, xp, neg), axis=0, keepdims=True)
    m1 = jnp.max(jnp.where(bcol == 1.0, xp, neg), axis=0, keepdims=True)
    h = jnp.concatenate([m0, m1], axis=0)

    def bn_relu(z, g, be):
        mu = jnp.mean(z, axis=0, keepdims=True)
        va = jnp.mean((z - mu) ** 2, axis=0, keepdims=True)
        return jnp.maximum((z - mu) / jnp.sqrt(va + 1e-5) * g + be, 0.0)

    h = bn_relu(jnp.dot(h, w1_ref[...], preferred_element_type=f32) + b1_ref[...],
                g1_ref[...], e1_ref[...])
    h = bn_relu(jnp.dot(h, w2_ref[...], preferred_element_type=f32) + b2_ref[...],
                g2_ref[...], e2_ref[...])
    o = jnp.dot(h, w3_ref[...], preferred_element_type=f32) + b3_ref[...]
    o = o - jnp.max(o, axis=1, keepdims=True)
    o_ref[...] = o - jnp.log(jnp.sum(jnp.exp(o), axis=1, keepdims=True))


def _head(x, bcol, p):
    args = (x, bcol, p['w_pool'], p['b_pool'].reshape(1, -1),
            p['w_f1'], p['b_f1'].reshape(1, -1),
            p['g_f1'].reshape(1, -1), p['be_f1'].reshape(1, -1),
            p['w_f2'], p['b_f2'].reshape(1, -1),
            p['g_f2'].reshape(1, -1), p['be_f2'].reshape(1, -1),
            p['w_f3'], p['b_f3'].reshape(1, -1))
    return pl.pallas_call(
        _head_body,
        out_shape=jax.ShapeDtypeStruct((2, 10), jnp.float32),
    )(*args)


def _round128(v):
    return (v + 127) // 128 * 128


def kernel(pos, batch, params):
    p = params
    x = pos
    bf = batch.astype(jnp.float32)
    for i, (din, dout) in enumerate(_BLOCKS):
        n = x.shape[0]
        nq = n // _DEC
        d4 = dout // 4
        d2 = dout // 2
        posp = _pad_lanes(pos, 8)
        pqk = posp.at[:, 3].set(bf)[::_DEC]
        ptk = posp.at[:, 3].set(bf).T

        dinp = max(8, din)
        xw = _pad_lanes(x, dinp)
        w1 = _pad_rows(p['w1_%d' % i], dinp)

        w1t = _round128(8 + d4)
        col, table1 = _knn_h1(pqk, ptk, xw, posp, w1,
                              p['b1_%d' % i].reshape(1, -1), w1t)
        e1 = _sc_gather(table1, col.reshape(-1))
        wenc = _pad_rows(p['wenc1_%d' % i], 16)
        a1 = _lfa(e1, posp[:nq],
                  wenc, p['benc1_%d' % i].reshape(1, -1),
                  p['watt1_%d' % i], p['batt1_%d' % i].reshape(1, -1))

        w2t = _round128(8 + d2)
        table2 = _pad_lanes(
            jnp.concatenate([posp, _pad_rows(a1, n)], axis=1), w2t)
        col2 = col[::_DEC].reshape(-1)
        e2 = _sc_gather(table2, col2)
        wenc = _pad_rows(p['wenc2_%d' % i], 16)
        pq2 = posp[::_DEC][:nq // _DEC]
        a2q = _lfa(e2, pq2,
                   wenc, p['benc2_%d' % i].reshape(1, -1),
                   p['watt2_%d' % i], p['batt2_%d' % i].reshape(1, -1))

        a2 = _pad_rows(a2q, nq)
        x4 = xw[::_DEC]
        wsc = _pad_rows(p['wsc_%d' % i], dinp)
        x = _block_out(a2, x4, p['w2_%d' % i], p['b2_%d' % i].reshape(1, -1),
                       wsc, p['bsc_%d' % i].reshape(1, -1))
        pos = pos[::_DEC]
        batch = batch[::_DEC]
        bf = bf[::_DEC]

    bcol = bf.reshape(-1, 1)
    return _head(x, bcol, p)


# speedup vs baseline: 5.0979x; 1.0307x over previous
import functools

import jax
import jax.numpy as jnp
from jax import lax
from jax.experimental import pallas as pl
from jax.experimental.pallas import tpu as pltpu
from jax.experimental.pallas import tpu_sc as plsc

_DEC = 4
_K = 16
_BLOCKS = [(3, 32), (32, 128), (128, 256), (256, 512)]
_BIG = 1e30


def _leaky(x, s):
    return jnp.where(x >= 0, x, s * x)


def _pad_lanes(x, w):
    if x.shape[1] == w:
        return x
    return jnp.pad(x, ((0, 0), (0, w - x.shape[1])))


def _pad_rows(x, n):
    if x.shape[0] == n:
        return x
    return jnp.pad(x, ((0, n - x.shape[0]), (0, 0)))


def _knn_h1_body(pq_ref, pt_ref, x_ref, posp_ref, w1_ref, b1_ref,
                 col_ref, tab_ref, *, n, w1t):
    pq = pq_ref[...]
    pt = pt_ref[...]
    d = ((pq[:, 0:1] - pt[0:1, :]) ** 2
         + (pq[:, 1:2] - pt[1:2, :]) ** 2
         + (pq[:, 2:3] - pt[2:3, :]) ** 2)
    d = jnp.where(pq[:, 3:4] != pt[3:4, :], _BIG, d)
    iota = lax.broadcasted_iota(jnp.int32, d.shape, 1)
    liota = lax.broadcasted_iota(jnp.int32, (d.shape[0], _K), 1)
    acc = jnp.zeros((d.shape[0], _K), jnp.int32)
    for t in range(_K):
        am = jnp.argmin(d, axis=1).astype(jnp.int32)[:, None]
        acc = jnp.where(liota == t, am, acc)
        d = jnp.where(iota == am, jnp.inf, d)
    col_ref[...] = acc
    h1 = _leaky(
        jnp.dot(x_ref[...], w1_ref[...], preferred_element_type=jnp.float32)
        + b1_ref[...], 0.2)
    rows = h1.shape[0]
    pad = w1t - 8 - h1.shape[1]
    tab_ref[...] = jnp.concatenate(
        [posp_ref[...], h1, jnp.zeros((rows, pad), jnp.float32)], axis=1)


def _knn_h1(pq, pt, x, posp, w1, b1, w1t):
    nq = pq.shape[0]
    n = pt.shape[1]
    tq = min(nq, 128)
    grid = nq // tq
    tr = n // grid
    d4 = w1.shape[1]
    full = lambda a: pl.BlockSpec(a.shape, lambda i: tuple(0 for _ in a.shape))
    return pl.pallas_call(
        functools.partial(_knn_h1_body, n=n, w1t=w1t),
        grid=(grid,),
        in_specs=[
            pl.BlockSpec((tq, 8), lambda i: (i, 0)),
            pl.BlockSpec((8, n), lambda i: (0, 0)),
            pl.BlockSpec((tr, x.shape[1]), lambda i: (i, 0)),
            pl.BlockSpec((tr, 8), lambda i: (i, 0)),
            full(w1), full(b1),
        ],
        out_specs=[
            pl.BlockSpec((tq, _K), lambda i: (i, 0)),
            pl.BlockSpec((tr, w1t), lambda i: (i, 0)),
        ],
        out_shape=[
            jax.ShapeDtypeStruct((nq, _K), jnp.int32),
            jax.ShapeDtypeStruct((n, w1t), jnp.float32),
        ],
    )(pq, pt, x, posp, w1, b1)


def _sc_gather(table, idx):
    b = idx.shape[0]
    d = table.shape[1]
    info = plsc.get_sparse_core_info()
    ncores = info.num_cores
    nw_max = info.num_cores * info.num_subcores
    nw = min(nw_max, b // 8)
    c = b // nw
    s = min(c, 128)
    nloop = c // s
    mesh = plsc.VectorSubcoreMesh(core_axis_name="c", subcore_axis_name="s")

    @functools.partial(
        pl.kernel, mesh=mesh,
        out_type=jax.ShapeDtypeStruct((b, d), jnp.float32),
        scratch_types=[
            pltpu.VMEM((c,), jnp.int32),
            pltpu.VMEM((s, d), jnp.float32),
            pltpu.VMEM((s, d), jnp.float32),
            pltpu.SemaphoreType.DMA,
            pltpu.SemaphoreType.DMA,
        ],
    )
    def k(table_hbm, idx_hbm, out_hbm, idx_v, row0, row1, sem0, sem1):
        wid = lax.axis_index("s") * ncores + lax.axis_index("c")
        bufs = (row0, row1)
        sems = (sem0, sem1)

        @pl.when(wid < nw)
        def _():
            base = wid * c
            pltpu.sync_copy(idx_hbm.at[pl.ds(base, c)], idx_v)
            pend = [None, None]
            pend[0] = pltpu.async_copy(
                table_hbm.at[idx_v.at[pl.ds(0, s)]], bufs[0], sems[0])
            for j in range(nloop):
                cur = j % 2
                nxt = (j + 1) % 2
                if j + 1 < nloop:
                    pend[nxt] = pltpu.async_copy(
                        table_hbm.at[idx_v.at[pl.ds((j + 1) * s, s)]],
                        bufs[nxt], sems[nxt])
                pend[cur].wait()
                pltpu.sync_copy(bufs[cur], out_hbm.at[pl.ds(base + j * s, s)])

    return k(table, idx)


def _lfa_body(e_ref, pq_ref, wenc_ref, benc_ref,
              watt_ref, batt_ref, o_ref, *, dj):
    e = e_ref[...]
    te = e.shape[0]
    tq = pq_ref.shape[0]
    f32 = jnp.float32
    hi = lax.Precision.HIGHEST
    r_eq = lax.broadcasted_iota(jnp.int32, (te, tq), 0) // _K
    r_q = lax.broadcasted_iota(jnp.int32, (te, tq), 1)
    rmat = (r_eq == r_q).astype(f32)
    pos_i = jnp.dot(rmat, pq_ref[...], preferred_element_type=f32, precision=hi)
    pos_j = e[:, 0:8]
    x_j = e[:, 8:8 + dj]
    dist = pos_j - pos_i
    euclid = jnp.sum(jnp.sqrt(dist * dist), axis=1, keepdims=True)
    rel = jnp.concatenate(
        [pos_i[:, 0:3], pos_j[:, 0:3], dist[:, 0:3], euclid,
         jnp.zeros((te, 6), f32)], axis=1)
    enc = jnp.dot(rel, wenc_ref[...], preferred_element_type=f32) + benc_ref[...]
    out1 = jnp.concatenate([x_j, enc], axis=1)
    lo = jnp.dot(out1, watt_ref[...], preferred_element_type=f32) + batt_ref[...]
    lo = lo - jnp.max(lo, axis=1, keepdims=True)
    p = jnp.exp(lo)
    att = p / jnp.sum(p, axis=1, keepdims=True)
    msg = att * out1
    r2_q = lax.broadcasted_iota(jnp.int32, (tq, te), 0)
    r2_e = lax.broadcasted_iota(jnp.int32, (tq, te), 1) // _K
    r2 = (r2_q == r2_e).astype(f32)
    o_ref[...] = jnp.dot(r2, msg, preferred_element_type=f32, precision=hi)


def _lfa(e, pq, wenc, benc, watt, batt):
    nq = pq.shape[0]
    w = e.shape[1]
    dj = watt.shape[0] - wenc.shape[1]
    d1 = watt.shape[0]
    tq = min(nq, 128)
    te = tq * _K
    grid = nq // tq
    full = lambda a: pl.BlockSpec(a.shape, lambda i: tuple(0 for _ in a.shape))
    return pl.pallas_call(
        functools.partial(_lfa_body, dj=dj),
        grid=(grid,),
        in_specs=[
            pl.BlockSpec((te, w), lambda i: (i, 0)),
            pl.BlockSpec((tq, 8), lambda i: (i, 0)),
            full(wenc), full(benc), full(watt), full(batt),
        ],
        out_specs=pl.BlockSpec((tq, d1), lambda i: (i, 0)),
        out_shape=jax.ShapeDtypeStruct((nq, d1), jnp.float32),
    )(e, pq, wenc, benc, watt, batt)


def _edge_col(col_f, rmat, te):
    rep = jnp.dot(rmat, col_f, preferred_element_type=jnp.float32,
                  precision=lax.Precision.HIGHEST)
    tsel = (lax.broadcasted_iota(jnp.int32, (te, _K), 1)
            == lax.broadcasted_iota(jnp.int32, (te, _K), 0) % _K)
    return jnp.sum(jnp.where(tsel, rep, 0.0), axis=1, keepdims=True)


def _lfa_math(pos_j, x_j, pos_i, wenc, benc, watt, batt):
    f32 = jnp.float32
    hi = lax.Precision.HIGHEST
    te = pos_j.shape[0]
    dist = pos_j - pos_i
    euclid = jnp.sum(jnp.sqrt(dist * dist), axis=1, keepdims=True)
    rel = jnp.concatenate(
        [pos_i[:, 0:3], pos_j[:, 0:3], dist[:, 0:3], euclid,
         jnp.zeros((te, 6), f32)], axis=1)
    enc = jnp.dot(rel, wenc, preferred_element_type=f32) + benc
    out1 = jnp.concatenate([x_j, enc], axis=1)
    lo = jnp.dot(out1, watt, preferred_element_type=f32) + batt
    lo = lo - jnp.max(lo, axis=1, keepdims=True)
    pr = jnp.exp(lo)
    att = pr / jnp.sum(pr, axis=1, keepdims=True)
    del hi
    return att * out1


def _block_fused_body(pq_ref, pt_ref, x_ref, posp_ref, w1_ref, b1_ref,
                      wenc1_ref, benc1_ref, watt1_ref, batt1_ref,
                      wenc2_ref, benc2_ref, watt2_ref, batt2_ref,
                      w2_ref, b2_ref, wsc_ref, bsc_ref, o_ref, *, n):
    f32 = jnp.float32
    hi = lax.Precision.HIGHEST
    nq = n // _DEC
    nq4 = nq // _DEC
    pq = pq_ref[...]
    pt = pt_ref[...]
    d = ((pq[:, 0:1] - pt[0:1, :]) ** 2
         + (pq[:, 1:2] - pt[1:2, :]) ** 2
         + (pq[:, 2:3] - pt[2:3, :]) ** 2)
    d = jnp.where(pq[:, 3:4] != pt[3:4, :], _BIG, d)
    iota = lax.broadcasted_iota(jnp.int32, d.shape, 1)
    liota = lax.broadcasted_iota(jnp.int32, (nq, _K), 1)
    acc = jnp.zeros((nq, _K), jnp.int32)
    for t in range(_K):
        am = jnp.argmin(d, axis=1).astype(jnp.int32)[:, None]
        acc = jnp.where(liota == t, am, acc)
        d = jnp.where(iota == am, jnp.inf, d)
    col_f = acc.astype(f32)

    posp = posp_ref[...]
    h1 = _leaky(
        jnp.dot(x_ref[...], w1_ref[...], preferred_element_type=f32)
        + b1_ref[...], 0.2)

    te1 = nq * _K
    r1 = (lax.broadcasted_iota(jnp.int32, (te1, nq), 0) // _K
          == lax.broadcasted_iota(jnp.int32, (te1, nq), 1)).astype(f32)
    colE1 = _edge_col(col_f, r1, te1).astype(jnp.int32)
    oh1 = (colE1 == lax.broadcasted_iota(jnp.int32, (te1, n), 1)).astype(f32)
    pos_j = jnp.dot(oh1, posp, preferred_element_type=f32, precision=hi)
    x_j = jnp.dot(oh1, h1, preferred_element_type=f32, precision=hi)
    pos_i = jnp.dot(r1, posp[0:nq], preferred_element_type=f32, precision=hi)
    msg1 = _lfa_math(pos_j, x_j, pos_i, wenc1_ref[...], benc1_ref[...],
                     watt1_ref[...], batt1_ref[...])
    r1t = (lax.broadcasted_iota(jnp.int32, (nq, te1), 0)
           == lax.broadcasted_iota(jnp.int32, (nq, te1), 1) // _K).astype(f32)
    a1 = jnp.dot(r1t, msg1, preferred_element_type=f32, precision=hi)

    te2 = nq4 * _K
    r2 = (_DEC * (lax.broadcasted_iota(jnp.int32, (te2, nq), 0) // _K)
          == lax.broadcasted_iota(jnp.int32, (te2, nq), 1)).astype(f32)
    colE2 = _edge_col(col_f, r2, te2).astype(jnp.int32)
    oh2 = (colE2 == lax.broadcasted_iota(jnp.int32, (te2, n), 1)).astype(f32)
    pos_j2 = jnp.dot(oh2, posp, preferred_element_type=f32, precision=hi)
    x_j2 = jnp.dot(oh2[:, 0:nq], a1, preferred_element_type=f32, precision=hi)
    pos_i2 = jnp.dot(r2, posp[0:nq], preferred_element_type=f32, precision=hi)
    msg2 = _lfa_math(pos_j2, x_j2, pos_i2, wenc2_ref[...], benc2_ref[...],
                     watt2_ref[...], batt2_ref[...])
    r2t = (lax.broadcasted_iota(jnp.int32, (nq4, te2), 0)
           == lax.broadcasted_iota(jnp.int32, (nq4, te2), 1) // _K).astype(f32)
    a2q = jnp.dot(r2t, msg2, preferred_element_type=f32, precision=hi)
    a2 = jnp.concatenate(
        [a2q, jnp.zeros((nq - nq4, a2q.shape[1]), f32)], axis=0)

    s4 = (_DEC * lax.broadcasted_iota(jnp.int32, (nq, n), 0)
          == lax.broadcasted_iota(jnp.int32, (nq, n), 1)).astype(f32)
    x4 = jnp.dot(s4, x_ref[...], preferred_element_type=f32, precision=hi)
    h2 = _leaky(jnp.dot(a2, w2_ref[...], preferred_element_type=f32)
                + b2_ref[...], 0.2)
    sc = _leaky(jnp.dot(x4, wsc_ref[...], preferred_element_type=f32)
                + bsc_ref[...], 0.2)
    o_ref[...] = _leaky(h2 + sc, 0.01)


def _block_fused(pq, pt, x, posp, p, i, dout):
    n = pt.shape[1]
    nq = n // _DEC
    return pl.pallas_call(
        functools.partial(_block_fused_body, n=n),
        out_shape=jax.ShapeDtypeStruct((nq, dout), jnp.float32),
    )(pq, pt, x, posp,
      _pad_rows(p['w1_%d' % i], x.shape[1]), p['b1_%d' % i].reshape(1, -1),
      _pad_rows(p['wenc1_%d' % i], 16), p['benc1_%d' % i].reshape(1, -1),
      p['watt1_%d' % i], p['batt1_%d' % i].reshape(1, -1),
      _pad_rows(p['wenc2_%d' % i], 16), p['benc2_%d' % i].reshape(1, -1),
      p['watt2_%d' % i], p['batt2_%d' % i].reshape(1, -1),
      p['w2_%d' % i], p['b2_%d' % i].reshape(1, -1),
      _pad_rows(p['wsc_%d' % i], x.shape[1]), p['bsc_%d' % i].reshape(1, -1))


def _out_body(a2_ref, x4_ref, w2_ref, b2_ref, wsc_ref, bsc_ref, o_ref):
    f32 = jnp.float32
    h2 = _leaky(jnp.dot(a2_ref[...], w2_ref[...], preferred_element_type=f32)
                + b2_ref[...], 0.2)
    sc = _leaky(jnp.dot(x4_ref[...], wsc_ref[...], preferred_element_type=f32)
                + bsc_ref[...], 0.2)
    o_ref[...] = _leaky(h2 + sc, 0.01)


def _block_out(a2, x4, w2, b2, wsc, bsc):
    m = a2.shape[0]
    return pl.pallas_call(
        _out_body,
        out_shape=jax.ShapeDtypeStruct((m, w2.shape[1]), jnp.float32),
    )(a2, x4, w2, b2, wsc, bsc)


def _head_body(x_ref, bcol_ref, wp_ref, bp_ref, w1_ref, b1_ref, g1_ref,
               e1_ref, w2_ref, b2_ref, g2_ref, e2_ref, w3_ref, b3_ref, o_ref):
    f32 = jnp.float32
    xp = jnp.dot(x_ref[...], wp_ref[...], preferred_element_type=f32) + bp_ref[...]
    bcol = bcol_ref[...]
    neg = jnp.float32(-jnp.inf)
    m0 = jnp.max(jnp.where(bcol == 0.0, xp, neg), axis=0, keepdims=True)
    m1 = jnp.max(jnp.where(bcol == 1.0, xp, neg), axis=0, keepdims=True)
    h = jnp.concatenate([m0, m1], axis=0)

    def bn_relu(z, g, be):
        mu = jnp.mean(z, axis=0, keepdims=True)
        va = jnp.mean((z - mu) ** 2, axis=0, keepdims=True)
        return jnp.maximum((z - mu) / jnp.sqrt(va + 1e-5) * g + be, 0.0)

    h = bn_relu(jnp.dot(h, w1_ref[...], preferred_element_type=f32) + b1_ref[...],
                g1_ref[...], e1_ref[...])
    h = bn_relu(jnp.dot(h, w2_ref[...], preferred_element_type=f32) + b2_ref[...],
                g2_ref[...], e2_ref[...])
    o = jnp.dot(h, w3_ref[...], preferred_element_type=f32) + b3_ref[...]
    o = o - jnp.max(o, axis=1, keepdims=True)
    o_ref[...] = o - jnp.log(jnp.sum(jnp.exp(o), axis=1, keepdims=True))


def _head(x, bcol, p):
    args = (x, bcol, p['w_pool'], p['b_pool'].reshape(1, -1),
            p['w_f1'], p['b_f1'].reshape(1, -1),
            p['g_f1'].reshape(1, -1), p['be_f1'].reshape(1, -1),
            p['w_f2'], p['b_f2'].reshape(1, -1),
            p['g_f2'].reshape(1, -1), p['be_f2'].reshape(1, -1),
            p['w_f3'], p['b_f3'].reshape(1, -1))
    return pl.pallas_call(
        _head_body,
        out_shape=jax.ShapeDtypeStruct((2, 10), jnp.float32),
    )(*args)


def _round128(v):
    return (v + 127) // 128 * 128


def kernel(pos, batch, params):
    p = params
    x = pos
    bf = batch.astype(jnp.float32)
    for i, (din, dout) in enumerate(_BLOCKS):
        n = x.shape[0]
        nq = n // _DEC
        d4 = dout // 4
        d2 = dout // 2
        posp = _pad_lanes(pos, 8)
        pqk = posp.at[:, 3].set(bf)[::_DEC]
        ptk = posp.at[:, 3].set(bf).T

        dinp = max(8, din)
        xw = _pad_lanes(x, dinp)
        w1 = _pad_rows(p['w1_%d' % i], dinp)

        if n <= 512:
            x = _block_fused(pqk, ptk, xw, posp, p, i, dout)
            pos = pos[::_DEC]
            batch = batch[::_DEC]
            bf = bf[::_DEC]
            continue

        w1t = _round128(8 + d4)
        col, table1 = _knn_h1(pqk, ptk, xw, posp, w1,
                              p['b1_%d' % i].reshape(1, -1), w1t)
        e1 = _sc_gather(table1, col.reshape(-1))
        wenc = _pad_rows(p['wenc1_%d' % i], 16)
        a1 = _lfa(e1, posp[:nq],
                  wenc, p['benc1_%d' % i].reshape(1, -1),
                  p['watt1_%d' % i], p['batt1_%d' % i].reshape(1, -1))

        w2t = _round128(8 + d2)
        table2 = _pad_lanes(
            jnp.concatenate([posp, _pad_rows(a1, n)], axis=1), w2t)
        col2 = col[::_DEC].reshape(-1)
        e2 = _sc_gather(table2, col2)
        wenc = _pad_rows(p['wenc2_%d' % i], 16)
        pq2 = posp[::_DEC][:nq // _DEC]
        a2q = _lfa(e2, pq2,
                   wenc, p['benc2_%d' % i].reshape(1, -1),
                   p['watt2_%d' % i], p['batt2_%d' % i].reshape(1, -1))

        a2 = _pad_rows(a2q, nq)
        x4 = xw[::_DEC]
        wsc = _pad_rows(p['wsc_%d' % i], dinp)
        x = _block_out(a2, x4, p['w2_%d' % i], p['b2_%d' % i].reshape(1, -1),
                       wsc, p['bsc_%d' % i].reshape(1, -1))
        pos = pos[::_DEC]
        batch = batch[::_DEC]
        bf = bf[::_DEC]

    bcol = bf.reshape(-1, 1)
    return _head(x, bcol, p)
